# baseline (speedup 1.0000x reference)
import functools

import jax
import jax.numpy as jnp
import numpy as np
from jax import lax
from jax.experimental import pallas as pl
from jax.experimental.pallas import tpu as pltpu
from jax.experimental.pallas import tpu_sc as plsc

NC = 2
NS = 16
CHUNK = 128

_LOG2 = float(np.log(2.0))


def _k1_body(x_ref, w_ref, o_ref):
    o_ref[...] = jnp.dot(x_ref[...], w_ref[...], preferred_element_type=jnp.float32)


def _matmul_xw1(x_pad, W1, block_rows):
    n_pad, feat = x_pad.shape
    emb = W1.shape[1]
    grid = (n_pad // block_rows,)
    return pl.pallas_call(
        _k1_body,
        grid=grid,
        in_specs=[
            pl.BlockSpec((block_rows, feat), lambda i: (i, 0)),
            pl.BlockSpec((feat, emb), lambda i: (0, 0)),
        ],
        out_specs=pl.BlockSpec((block_rows, emb), lambda i: (i, 0)),
        out_shape=jax.ShapeDtypeStruct((n_pad, emb), jnp.float32),
    )(x_pad, W1)


def _sc_scatter(y_pad, src3, dst3, zeros_acc):
    n_acc, emb = y_pad.shape
    ch = src3.shape[1]
    rows_per_tile = n_acc // NS
    mesh = plsc.VectorSubcoreMesh(
        core_axis_name="c", subcore_axis_name="s", num_cores=NC, num_subcores=NS
    )

    @functools.partial(
        pl.kernel,
        out_type=jax.ShapeDtypeStruct((NC, n_acc, emb), jnp.float32),
        mesh=mesh,
        compiler_params=pltpu.CompilerParams(use_tc_tiling_on_sc=False),
        scratch_types=[
            pltpu.VMEM((ch, CHUNK), jnp.int32),
            pltpu.VMEM((ch, CHUNK), jnp.int32),
            pltpu.VMEM((CHUNK, emb), jnp.float32),
            pltpu.VMEM((CHUNK, emb), jnp.float32),
            pltpu.VMEM_SHARED((n_acc, emb), jnp.float32),
            pltpu.SemaphoreType.DMA,
            pltpu.SemaphoreType.DMA,
        ],
    )
    def k(y_hbm, src_hbm, dst_hbm, z_hbm, out_hbm, src_v, dst_v, rows0, rows1,
          acc, sem0, sem1):
        c = lax.axis_index("c")
        s = lax.axis_index("s")
        t = c * NS + s

        @pl.when(s == 0)
        def _():
            pltpu.sync_copy(z_hbm, acc)

        pltpu.sync_copy(src_hbm.at[t], src_v)
        pltpu.sync_copy(dst_hbm.at[t], dst_v)
        plsc.subcore_barrier()

        pltpu.async_copy(y_hbm.at[src_v.at[0]], rows0, sem0)

        def body(i, _):
            j = 2 * i
            pltpu.async_copy(y_hbm.at[src_v.at[j + 1]], rows1, sem1)
            pltpu.make_async_copy(y_hbm.at[src_v.at[j]], rows0, sem0).wait()
            pltpu.sync_copy(rows0, acc.at[dst_v.at[j]], add=True)

            @pl.when(j + 2 < ch)
            def _():
                pltpu.async_copy(y_hbm.at[src_v.at[j + 2]], rows0, sem0)

            pltpu.make_async_copy(y_hbm.at[src_v.at[j + 1]], rows1, sem1).wait()
            pltpu.sync_copy(rows1, acc.at[dst_v.at[j + 1]], add=True)
            return 0

        lax.fori_loop(0, ch // 2, body, 0, unroll=False)
        plsc.subcore_barrier()

        base = s * rows_per_tile
        pltpu.sync_copy(
            acc.at[pl.ds(base, rows_per_tile)],
            out_hbm.at[c].at[pl.ds(base, rows_per_tile)],
        )

    return k(y_pad, src3, dst3, zeros_acc)


def _ff_block(z, Wa, ba, Wb, bb, Wc, bc, Ws, bs):
    t = jnp.maximum(jnp.dot(z, Wa, preferred_element_type=jnp.float32) + ba, 0.0)
    t = jnp.maximum(jnp.dot(t, Wb, preferred_element_type=jnp.float32) + bb, 0.0)
    t = jnp.maximum(jnp.dot(t, Wc, preferred_element_type=jnp.float32) + bc, 0.0)
    return t + jnp.dot(z, Ws, preferred_element_type=jnp.float32) + bs


def _k2_body(n_graphs, p0_ref, p1_ref, y_ref, gid_ref, w2_ref, b2_ref, b1_ref,
             lwa_ref, lba_ref, lwb_ref, lbb_ref, lwc_ref, lbc_ref, lws_ref,
             lbs_ref, lenc_ref, gh_ref):
    h1 = jnp.maximum(p0_ref[...] + p1_ref[...] + y_ref[...] + b1_ref[...], 0.0)
    h2 = jnp.dot(h1, w2_ref[...], preferred_element_type=jnp.float32) + b2_ref[...]
    lh = jnp.maximum(h2, 0.0)
    lenc_ref[...] = _ff_block(lh, lwa_ref[...], lba_ref[...], lwb_ref[...],
                              lbb_ref[...], lwc_ref[...], lbc_ref[...],
                              lws_ref[...], lbs_ref[...])
    gid = gid_ref[...]
    rows = gid.shape[0]
    onehot = (gid == lax.broadcasted_iota(jnp.int32, (rows, n_graphs), 1)
              ).astype(jnp.float32)
    contrib = lax.dot_general(onehot, lh, (((0,), (0,)), ((), ())),
                              preferred_element_type=jnp.float32)

    @pl.when(pl.program_id(0) == 0)
    def _():
        gh_ref[...] = contrib

    @pl.when(pl.program_id(0) != 0)
    def _():
        gh_ref[...] += contrib


def _k2(p0, p1, y_pad, gid2, W2, b2r, b1r, lWa, lbar, lWb, lbbr, lWc, lbcr,
        lWs, lbsr, n_graphs, block_rows):
    n_pad, emb = y_pad.shape
    grid = (n_pad // block_rows,)
    row_spec = pl.BlockSpec((block_rows, emb), lambda i: (i, 0))
    w_spec = pl.BlockSpec((emb, emb), lambda i: (0, 0))
    b_spec = pl.BlockSpec((1, emb), lambda i: (0, 0))
    return pl.pallas_call(
        functools.partial(_k2_body, n_graphs),
        grid=grid,
        in_specs=[
            row_spec, row_spec, row_spec,
            pl.BlockSpec((block_rows, 1), lambda i: (i, 0)),
            w_spec, b_spec, b_spec,
            w_spec, b_spec, w_spec, b_spec, w_spec, b_spec, w_spec, b_spec,
        ],
        out_specs=[
            row_spec,
            pl.BlockSpec((n_graphs, emb), lambda i: (0, 0)),
        ],
        out_shape=[
            jax.ShapeDtypeStruct((n_pad, emb), jnp.float32),
            jax.ShapeDtypeStruct((n_graphs, emb), jnp.float32),
        ],
    )(p0, p1, y_pad, gid2, W2, b2r, b1r, lWa, lbar, lWb, lbbr, lWc, lbcr,
      lWs, lbsr)


def _k3_body(n_nodes, n_graphs, lenc_ref, gh_ref, gid_ref, gwa_ref, gba_ref,
             gwb_ref, gbb_ref, gwc_ref, gbc_ref, gws_ref, gbs_ref, out_ref,
             genc_scr):
    @pl.when(pl.program_id(0) == 0)
    def _():
        genc_scr[...] = _ff_block(gh_ref[...], gwa_ref[...], gba_ref[...],
                                  gwb_ref[...], gbb_ref[...], gwc_ref[...],
                                  gbc_ref[...], gws_ref[...], gbs_ref[...])
        out_ref[0, 0] = 0.0

    genc = genc_scr[...]
    res = lax.dot_general(lenc_ref[...], genc, (((1,), (1,)), ((), ())),
                          preferred_element_type=jnp.float32)
    gid = gid_ref[...]
    rows = gid.shape[0]
    cols = lax.broadcasted_iota(jnp.int32, (rows, n_graphs), 1)
    pos = gid == cols
    neg = jnp.logical_and(gid >= 0, jnp.logical_not(pos))
    soft = jnp.log1p(jnp.exp(-jnp.abs(res)))
    sp_pos_arg = jnp.maximum(-res, 0.0) + soft
    sp_neg_arg = jnp.maximum(res, 0.0) + soft
    neg_c = jnp.where(neg, sp_neg_arg - _LOG2, 0.0)
    pos_c = jnp.where(pos, _LOG2 - sp_pos_arg, 0.0)
    val = (jnp.sum(neg_c) / (n_nodes * (n_graphs - 1))
           - jnp.sum(pos_c) / n_nodes)
    out_ref[0, 0] += val


def _k3(lenc, gh, gid2, gWa, gbar, gWb, gbbr, gWc, gbcr, gWs, gbsr, n_nodes,
        n_graphs, block_rows):
    n_pad, emb = lenc.shape
    grid = (n_pad // block_rows,)
    w_spec = pl.BlockSpec((emb, emb), lambda i: (0, 0))
    b_spec = pl.BlockSpec((1, emb), lambda i: (0, 0))
    return pl.pallas_call(
        functools.partial(_k3_body, n_nodes, n_graphs),
        grid=grid,
        in_specs=[
            pl.BlockSpec((block_rows, emb), lambda i: (i, 0)),
            pl.BlockSpec((n_graphs, emb), lambda i: (0, 0)),
            pl.BlockSpec((block_rows, 1), lambda i: (i, 0)),
            w_spec, b_spec, w_spec, b_spec, w_spec, b_spec, w_spec, b_spec,
        ],
        out_specs=pl.BlockSpec(memory_space=pltpu.SMEM),
        out_shape=jax.ShapeDtypeStruct((1, 1), jnp.float32),
        scratch_shapes=[pltpu.VMEM((n_graphs, emb), jnp.float32)],
    )(lenc, gh, gid2, gWa, gbar, gWb, gbbr, gWc, gbcr, gWs, gbsr)


def kernel(x, edge_index, graph_id, W1, b1, W2, b2, lWa, lba, lWb, lbb, lWc,
           lbc, lWs, lbs, gWa, gba, gWb, gbb, gWc, gbc, gWs, gbs):
    n_nodes, feat = x.shape
    emb = W1.shape[1]
    n_graphs = 128
    n_edges = edge_index.shape[1]

    block_rows = 512
    n_pad = ((n_nodes + block_rows - 1) // block_rows) * block_rows

    per = NC * NS * CHUNK
    ch = (n_edges + per - 1) // per
    ch += ch % 2
    e_pad = NC * NS * ch * CHUNK

    x_pad = jnp.concatenate(
        [x, jnp.zeros((n_pad - n_nodes, feat), jnp.float32)], axis=0)
    gid_pad = jnp.concatenate(
        [graph_id, jnp.full((n_pad - n_nodes,), -1, jnp.int32)])[:, None]

    src = edge_index[0]
    dst = edge_index[1]
    pad_idx = jnp.full((e_pad - n_edges,), n_nodes, jnp.int32)
    src3 = jnp.concatenate([src, pad_idx]).reshape(NC * NS, ch, CHUNK)
    dst3 = jnp.concatenate([dst, pad_idx]).reshape(NC * NS, ch, CHUNK)

    y_pad = _matmul_xw1(x_pad, W1, block_rows)
    zeros_acc = jnp.zeros((n_pad, emb), jnp.float32)
    partials = _sc_scatter(y_pad, src3, dst3, zeros_acc)

    b1r = b1[None, :]
    b2r = b2[None, :]
    lenc, gh = _k2(partials[0], partials[1], y_pad, gid_pad, W2, b2r, b1r,
                   lWa, lba[None, :], lWb, lbb[None, :], lWc, lbc[None, :],
                   lWs, lbs[None, :], n_graphs, block_rows)
    out = _k3(lenc, gh, gid_pad, gWa, gba[None, :], gWb, gbb[None, :],
              gWc, gbc[None, :], gWs, gbs[None, :], n_nodes, n_graphs,
              block_rows)
    return out[0, 0]


# speedup vs baseline: 1.6395x; 1.6395x over previous
import functools

import jax
import jax.numpy as jnp
import numpy as np
from jax import lax
from jax.experimental import pallas as pl
from jax.experimental.pallas import tpu as pltpu
from jax.experimental.pallas import tpu_sc as plsc

NC = 2
NS = 16
CHUNK = 125

_LOG2 = float(np.log(2.0))


def _k1_body(x_ref, w_ref, o_ref):
    o_ref[...] = jnp.dot(x_ref[...], w_ref[...], preferred_element_type=jnp.float32)


def _matmul_xw1(x, W1, block_rows):
    n, feat = x.shape
    emb = W1.shape[1]
    grid = (n // block_rows,)
    return pl.pallas_call(
        _k1_body,
        grid=grid,
        in_specs=[
            pl.BlockSpec((block_rows, feat), lambda i: (i, 0)),
            pl.BlockSpec((feat, emb), lambda i: (0, 0)),
        ],
        out_specs=pl.BlockSpec((block_rows, emb), lambda i: (i, 0)),
        out_shape=jax.ShapeDtypeStruct((n, emb), jnp.float32),
    )(x, W1)


def _sc_scatter(y, ei3, zeros_acc):
    n_acc, emb = y.shape
    ch = ei3.shape[1]
    rows_per_tile = n_acc // NS
    mesh = plsc.VectorSubcoreMesh(
        core_axis_name="c", subcore_axis_name="s", num_cores=NC, num_subcores=NS
    )

    @functools.partial(
        pl.kernel,
        out_type=jax.ShapeDtypeStruct((NC, n_acc, emb), jnp.float32),
        mesh=mesh,
        compiler_params=pltpu.CompilerParams(use_tc_tiling_on_sc=False),
        scratch_types=[
            pltpu.VMEM((ch // NC // NS, CHUNK), jnp.int32),
            pltpu.VMEM((ch // NC // NS, CHUNK), jnp.int32),
            pltpu.VMEM((CHUNK, emb), jnp.float32),
            pltpu.VMEM_SHARED((n_acc, emb), jnp.float32),
            pltpu.SemaphoreType.DMA,
        ],
    )
    def k(y_hbm, ei_hbm, z_hbm, out_hbm, src_v, dst_v, rows0, acc, sem0):
        c = lax.axis_index("c")
        s = lax.axis_index("s")
        t = c * NS + s
        ch_t = ch // NC // NS

        @pl.when(s == 0)
        def _():
            pltpu.sync_copy(z_hbm, acc)

        pltpu.sync_copy(ei_hbm.at[0].at[pl.ds(t * ch_t, ch_t)], src_v)
        pltpu.sync_copy(ei_hbm.at[1].at[pl.ds(t * ch_t, ch_t)], dst_v)
        plsc.subcore_barrier()

        def body(j, _):
            pltpu.async_copy(y_hbm.at[src_v.at[j]], rows0, sem0).wait()
            pltpu.sync_copy(rows0, acc.at[dst_v.at[j]], add=True)
            return 0

        lax.fori_loop(0, ch_t, body, 0, unroll=False)
        plsc.subcore_barrier()

        base = s * rows_per_tile
        pltpu.sync_copy(
            acc.at[pl.ds(base, rows_per_tile)],
            out_hbm.at[c].at[pl.ds(base, rows_per_tile)],
        )

    return k(y, ei3, zeros_acc)


def _ff_block(z, Wa, ba, Wb, bb, Wc, bc, Ws, bs):
    t = jnp.maximum(jnp.dot(z, Wa, preferred_element_type=jnp.float32) + ba, 0.0)
    t = jnp.maximum(jnp.dot(t, Wb, preferred_element_type=jnp.float32) + bb, 0.0)
    t = jnp.maximum(jnp.dot(t, Wc, preferred_element_type=jnp.float32) + bc, 0.0)
    return t + jnp.dot(z, Ws, preferred_element_type=jnp.float32) + bs


def _k2_body(n_graphs, p0_ref, p1_ref, y_ref, gid_ref, w2_ref, b2_ref, b1_ref,
             lwa_ref, lba_ref, lwb_ref, lbb_ref, lwc_ref, lbc_ref, lws_ref,
             lbs_ref, lenc_ref, gh_ref):
    h1 = jnp.maximum(p0_ref[...] + p1_ref[...] + y_ref[...] + b1_ref[...], 0.0)
    h2 = jnp.dot(h1, w2_ref[...], preferred_element_type=jnp.float32) + b2_ref[...]
    lh = jnp.maximum(h2, 0.0)
    lenc_ref[...] = _ff_block(lh, lwa_ref[...], lba_ref[...], lwb_ref[...],
                              lbb_ref[...], lwc_ref[...], lbc_ref[...],
                              lws_ref[...], lbs_ref[...])
    gid = gid_ref[...]
    rows = gid.shape[0]
    onehot = (gid == lax.broadcasted_iota(jnp.int32, (rows, n_graphs), 1)
              ).astype(jnp.float32)
    contrib = lax.dot_general(onehot, lh, (((0,), (0,)), ((), ())),
                              preferred_element_type=jnp.float32)

    @pl.when(pl.program_id(0) == 0)
    def _():
        gh_ref[...] = contrib

    @pl.when(pl.program_id(0) != 0)
    def _():
        gh_ref[...] += contrib


def _k2(p0, p1, y, gid2, W2, b2r, b1r, lWa, lbar, lWb, lbbr, lWc, lbcr,
        lWs, lbsr, n_graphs, block_rows):
    n, emb = y.shape
    grid = (n // block_rows,)
    row_spec = pl.BlockSpec((block_rows, emb), lambda i: (i, 0))
    w_spec = pl.BlockSpec((emb, emb), lambda i: (0, 0))
    b_spec = pl.BlockSpec((1, emb), lambda i: (0, 0))
    return pl.pallas_call(
        functools.partial(_k2_body, n_graphs),
        grid=grid,
        in_specs=[
            row_spec, row_spec, row_spec,
            pl.BlockSpec((block_rows, 1), lambda i: (i, 0)),
            w_spec, b_spec, b_spec,
            w_spec, b_spec, w_spec, b_spec, w_spec, b_spec, w_spec, b_spec,
        ],
        out_specs=[
            row_spec,
            pl.BlockSpec((n_graphs, emb), lambda i: (0, 0)),
        ],
        out_shape=[
            jax.ShapeDtypeStruct((n, emb), jnp.float32),
            jax.ShapeDtypeStruct((n_graphs, emb), jnp.float32),
        ],
    )(p0, p1, y, gid2, W2, b2r, b1r, lWa, lbar, lWb, lbbr, lWc, lbcr,
      lWs, lbsr)


def _k3_body(n_nodes, n_graphs, lenc_ref, gh_ref, gid_ref, gwa_ref, gba_ref,
             gwb_ref, gbb_ref, gwc_ref, gbc_ref, gws_ref, gbs_ref, out_ref,
             genc_scr):
    @pl.when(pl.program_id(0) == 0)
    def _():
        genc_scr[...] = _ff_block(gh_ref[...], gwa_ref[...], gba_ref[...],
                                  gwb_ref[...], gbb_ref[...], gwc_ref[...],
                                  gbc_ref[...], gws_ref[...], gbs_ref[...])
        out_ref[0, 0] = 0.0

    genc = genc_scr[...]
    res = lax.dot_general(lenc_ref[...], genc, (((1,), (1,)), ((), ())),
                          preferred_element_type=jnp.float32)
    gid = gid_ref[...]
    rows = gid.shape[0]
    cols = lax.broadcasted_iota(jnp.int32, (rows, n_graphs), 1)
    pos = gid == cols
    soft = jnp.log1p(jnp.exp(-jnp.abs(res)))
    sp_m = jnp.maximum(-res, 0.0) + soft
    sp_p = jnp.maximum(res, 0.0) + soft
    neg_c = jnp.where(pos, 0.0, sp_p - _LOG2)
    pos_c = jnp.where(pos, _LOG2 - sp_m, 0.0)
    val = (jnp.sum(neg_c) / (n_nodes * (n_graphs - 1))
           - jnp.sum(pos_c) / n_nodes)
    out_ref[0, 0] += val


def _k3(lenc, gh, gid2, gWa, gbar, gWb, gbbr, gWc, gbcr, gWs, gbsr, n_nodes,
        n_graphs, block_rows):
    n, emb = lenc.shape
    grid = (n // block_rows,)
    w_spec = pl.BlockSpec((emb, emb), lambda i: (0, 0))
    b_spec = pl.BlockSpec((1, emb), lambda i: (0, 0))
    return pl.pallas_call(
        functools.partial(_k3_body, n_nodes, n_graphs),
        grid=grid,
        in_specs=[
            pl.BlockSpec((block_rows, emb), lambda i: (i, 0)),
            pl.BlockSpec((n_graphs, emb), lambda i: (0, 0)),
            pl.BlockSpec((block_rows, 1), lambda i: (i, 0)),
            w_spec, b_spec, w_spec, b_spec, w_spec, b_spec, w_spec, b_spec,
        ],
        out_specs=pl.BlockSpec(memory_space=pltpu.SMEM),
        out_shape=jax.ShapeDtypeStruct((1, 1), jnp.float32),
        scratch_shapes=[pltpu.VMEM((n_graphs, emb), jnp.float32)],
    )(lenc, gh, gid2, gWa, gbar, gWb, gbbr, gWc, gbcr, gWs, gbsr)


def kernel(x, edge_index, graph_id, W1, b1, W2, b2, lWa, lba, lWb, lbb, lWc,
           lbc, lWs, lbs, gWa, gba, gWb, gbb, gWc, gbc, gWs, gbs):
    n_nodes, feat = x.shape
    emb = W1.shape[1]
    n_graphs = 128
    n_edges = edge_index.shape[1]

    block_rows = 2000

    n_chunks = n_edges // CHUNK
    ei3 = edge_index.reshape(2, n_chunks, CHUNK)

    y = _matmul_xw1(x, W1, block_rows)
    zeros_acc = jnp.zeros((n_nodes, emb), jnp.float32)
    partials = _sc_scatter(y, ei3, zeros_acc)

    gid2 = graph_id[:, None]
    lenc, gh = _k2(partials[0], partials[1], y, gid2, W2, b2[None, :],
                   b1[None, :], lWa, lba[None, :], lWb, lbb[None, :],
                   lWc, lbc[None, :], lWs, lbs[None, :], n_graphs, block_rows)
    out = _k3(lenc, gh, gid2, gWa, gba[None, :], gWb, gbb[None, :],
              gWc, gbc[None, :], gWs, gbs[None, :], n_nodes, n_graphs,
              block_rows)
    return out[0, 0]
